# Initial kernel scaffold; baseline (speedup 1.0000x reference)
import jax
import jax.numpy as jnp
from jax.experimental import pallas as pl


def kernel(padded_obj, positions):
    raise NotImplementedError("write your pallas kernel here")


# speedup vs baseline: 3.4559x; 3.4559x over previous
import jax
import jax.numpy as jnp
from jax import lax
from jax.experimental import pallas as pl
from jax.experimental.pallas import tpu as pltpu
from jax.experimental.pallas import tpu_sc as plsc

B = 4096
M = 74
N = 64
HALF = 5.0

NC = 2
NS = 16
NW = NC * NS
BPW = B // NW
L = 16
NVEC = N // L


def _floor_f32(x):
  t = x.astype(jnp.int32)
  return t - (t.astype(jnp.float32) > x).astype(jnp.int32)


def _patch_body(obj_hbm, px_hbm, py_hbm, out_hbm, img, outb, pxl, pyl):
  wid = lax.axis_index("s") * NC + lax.axis_index("c")
  base = wid * BPW
  pltpu.sync_copy(px_hbm.at[pl.ds(base, BPW)], pxl)
  pltpu.sync_copy(py_hbm.at[pl.ds(base, BPW)], pyl)
  iot = lax.iota(jnp.int32, 16)

  def batch_body(g, carry):
    b = base + g
    pltpu.sync_copy(obj_hbm.at[b], img)
    grp = (g // L) * L
    lane = g % L
    oh = iot == lane
    pxv = pxl[pl.ds(grp, L)] + HALF
    pyv = pyl[pl.ds(grp, L)] + HALF
    oxv = _floor_f32(pxv)
    oyv = _floor_f32(pyv)
    fxv = pxv - oxv.astype(jnp.float32)
    fyv = pyv - oyv.astype(jnp.float32)
    ox = jnp.sum(jnp.where(oh, oxv, 0))
    oy = jnp.sum(jnp.where(oh, oyv, 0))
    fx = jnp.sum(jnp.where(oh, fxv, 0.0))
    fy = jnp.sum(jnp.where(oh, fyv, 0.0))

    c0 = [jnp.clip(iot + (v * L) + ox, 0, M - 1) for v in range(NVEC)]
    c1 = [jnp.minimum(c + 1, M - 1) for c in c0]

    def row_body(i, carry2):
      r0 = jnp.clip(i + oy, 0, M - 1)
      r1 = jnp.minimum(r0 + 1, M - 1)
      r0v = jnp.zeros((L,), jnp.int32) + r0
      r1v = jnp.zeros((L,), jnp.int32) + r1
      rb0 = r0v * M
      rb1 = r1v * M
      for v in range(NVEC):
        v00 = plsc.load_gather(img, [rb0 + c0[v]])
        v01 = plsc.load_gather(img, [rb0 + c1[v]])
        v10 = plsc.load_gather(img, [rb1 + c0[v]])
        v11 = plsc.load_gather(img, [rb1 + c1[v]])
        top = v00 + fx * (v01 - v00)
        bot = v10 + fx * (v11 - v10)
        outb[pl.ds(i * N + v * L, L)] = top + fy * (bot - top)
      return carry2

    lax.fori_loop(0, N, row_body, 0)
    pltpu.sync_copy(outb, out_hbm.at[b])
    return carry

  lax.fori_loop(0, BPW, batch_body, 0)


@jax.jit
def _run(obj, px, py):
  kern = pl.kernel(
      _patch_body,
      out_type=jax.ShapeDtypeStruct((B, N * N), jnp.float32),
      mesh=plsc.VectorSubcoreMesh(core_axis_name="c", subcore_axis_name="s"),
      compiler_params=pltpu.CompilerParams(needs_layout_passes=False),
      scratch_types=[
          pltpu.VMEM((M * M,), jnp.float32),
          pltpu.VMEM((N * N,), jnp.float32),
          pltpu.VMEM((BPW,), jnp.float32),
          pltpu.VMEM((BPW,), jnp.float32),
      ],
  )
  return kern(obj, px, py)


def kernel(padded_obj, positions):
  obj = padded_obj.reshape(B, M * M)
  px = positions[:, 0]
  py = positions[:, 1]
  out = _run(obj, px, py)
  return out.reshape(B, N, N, 1)


# speedup vs baseline: 4.0189x; 1.1629x over previous
import jax
import jax.numpy as jnp
from jax import lax
from jax.experimental import pallas as pl
from jax.experimental.pallas import tpu as pltpu
from jax.experimental.pallas import tpu_sc as plsc

B = 4096
M = 74
N = 64
HALF = 5.0

NC = 2
NS = 16
NW = NC * NS
BPW = B // NW
L = 16
NVEC = N // L


def _floor_f32(x):
  t = x.astype(jnp.int32)
  return t - (t.astype(jnp.float32) > x).astype(jnp.int32)


def _patch_body(obj_hbm, px_hbm, py_hbm, out_hbm,
                img0, img1, ob0, ob1, pxl, pyl,
                is0, is1, os0, os1):
  wid = lax.axis_index("s") * NC + lax.axis_index("c")
  base = wid * BPW
  pltpu.sync_copy(px_hbm.at[pl.ds(base, BPW)], pxl)
  pltpu.sync_copy(py_hbm.at[pl.ds(base, BPW)], pyl)
  iot = lax.iota(jnp.int32, 16)

  pltpu.async_copy(obj_hbm.at[base], img0, is0)
  pltpu.async_copy(obj_hbm.at[base + 1], img1, is1)

  def compute_batch(g, img, ob):
    grp = (g // L) * L
    lane = g % L
    oh = iot == lane
    pxv = pxl[pl.ds(grp, L)] + HALF
    pyv = pyl[pl.ds(grp, L)] + HALF
    oxv = _floor_f32(pxv)
    oyv = _floor_f32(pyv)
    fxv = pxv - oxv.astype(jnp.float32)
    fyv = pyv - oyv.astype(jnp.float32)
    ox = jnp.sum(jnp.where(oh, oxv, 0))
    oy = jnp.sum(jnp.where(oh, oyv, 0))
    fx = jnp.sum(jnp.where(oh, fxv, 0.0))
    fy = jnp.sum(jnp.where(oh, fyv, 0.0))

    c0 = [jnp.clip(iot + (v * L) + ox, 0, M - 1) for v in range(NVEC)]
    c1 = [jnp.minimum(c + 1, M - 1) for c in c0]

    def row_body(i, carry2):
      r0 = jnp.clip(i + oy, 0, M - 1)
      r1 = jnp.minimum(r0 + 1, M - 1)
      rb0 = jnp.zeros((L,), jnp.int32) + r0 * M
      rb1 = jnp.zeros((L,), jnp.int32) + r1 * M
      for v in range(NVEC):
        v00 = plsc.load_gather(img, [rb0 + c0[v]])
        v01 = plsc.load_gather(img, [rb0 + c1[v]])
        v10 = plsc.load_gather(img, [rb1 + c0[v]])
        v11 = plsc.load_gather(img, [rb1 + c1[v]])
        top = v00 + fx * (v01 - v00)
        bot = v10 + fx * (v11 - v10)
        ob[pl.ds(i * N + v * L, L)] = top + fy * (bot - top)
      return carry2

    lax.fori_loop(0, N, row_body, 0)

  def pair_body(p, carry):
    for k, img, ob, isem, osem in ((0, img0, ob0, is0, os0),
                                   (1, img1, ob1, is1, os1)):
      g = 2 * p + k
      b = base + g
      pltpu.make_async_copy(obj_hbm.at[b], img, isem).wait()

      @pl.when(p > 0)
      def _():
        pltpu.make_async_copy(ob, out_hbm.at[b - 2], osem).wait()

      compute_batch(g, img, ob)
      pltpu.async_copy(ob, out_hbm.at[b], osem)

      @pl.when(p < BPW // 2 - 1)
      def _():
        pltpu.async_copy(obj_hbm.at[b + 2], img, isem)
    return carry

  lax.fori_loop(0, BPW // 2, pair_body, 0)
  pltpu.make_async_copy(ob0, out_hbm.at[base + BPW - 2], os0).wait()
  pltpu.make_async_copy(ob1, out_hbm.at[base + BPW - 1], os1).wait()


@jax.jit
def _run(obj, px, py):
  kern = pl.kernel(
      _patch_body,
      out_type=jax.ShapeDtypeStruct((B, N * N), jnp.float32),
      mesh=plsc.VectorSubcoreMesh(core_axis_name="c", subcore_axis_name="s"),
      compiler_params=pltpu.CompilerParams(needs_layout_passes=False),
      scratch_types=[
          pltpu.VMEM((M * M,), jnp.float32),
          pltpu.VMEM((M * M,), jnp.float32),
          pltpu.VMEM((N * N,), jnp.float32),
          pltpu.VMEM((N * N,), jnp.float32),
          pltpu.VMEM((BPW,), jnp.float32),
          pltpu.VMEM((BPW,), jnp.float32),
          pltpu.SemaphoreType.DMA,
          pltpu.SemaphoreType.DMA,
          pltpu.SemaphoreType.DMA,
          pltpu.SemaphoreType.DMA,
      ],
  )
  return kern(obj, px, py)


def kernel(padded_obj, positions):
  obj = padded_obj.reshape(B, M * M)
  px = positions[:, 0]
  py = positions[:, 1]
  out = _run(obj, px, py)
  return out.reshape(B, N, N, 1)


# speedup vs baseline: 5.8072x; 1.4450x over previous
import jax
import jax.numpy as jnp
from jax import lax
from jax.experimental import pallas as pl
from jax.experimental.pallas import tpu as pltpu
from jax.experimental.pallas import tpu_sc as plsc

B = 4096
M = 74
N = 64
HALF = 5.0

NC = 2
NS = 16
NW = NC * NS
BPW = B // NW
L = 16
NVEC = N // L


def _floor_f32(x):
  t = x.astype(jnp.int32)
  return t - (t.astype(jnp.float32) > x).astype(jnp.int32)


def _patch_body(obj_hbm, px_hbm, py_hbm, out_hbm,
                img0, img1, ob0, ob1, pxl, pyl,
                is0, is1, os0, os1):
  wid = lax.axis_index("s") * NC + lax.axis_index("c")
  base = wid * BPW
  pltpu.sync_copy(px_hbm.at[pl.ds(base, BPW)], pxl)
  pltpu.sync_copy(py_hbm.at[pl.ds(base, BPW)], pyl)
  iot = lax.iota(jnp.int32, 16)

  pltpu.async_copy(obj_hbm.at[base], img0, is0)
  pltpu.async_copy(obj_hbm.at[base + 1], img1, is1)

  def compute_batch(g, img, ob):
    grp = (g // L) * L
    lane = g % L
    oh = iot == lane
    pxv = pxl[pl.ds(grp, L)] + HALF
    pyv = pyl[pl.ds(grp, L)] + HALF
    oxv = _floor_f32(pxv)
    oyv = _floor_f32(pyv)
    fxv = pxv - oxv.astype(jnp.float32)
    fyv = pyv - oyv.astype(jnp.float32)
    ox = jnp.sum(jnp.where(oh, oxv, 0))
    oy = jnp.sum(jnp.where(oh, oyv, 0))
    fx = jnp.sum(jnp.where(oh, fxv, 0.0))
    fy = jnp.sum(jnp.where(oh, fyv, 0.0))

    c0 = [jnp.clip(iot + (v * L) + ox, 0, M - 1) for v in range(NVEC)]
    c1 = [jnp.minimum(c + 1, M - 1) for c in c0]

    @plsc.parallel_loop(0, N, 1, unroll=4)
    def row_body(i):
      r0 = jnp.clip(i + oy, 0, M - 1)
      r1 = jnp.minimum(r0 + 1, M - 1)
      rb0 = jnp.zeros((L,), jnp.int32) + r0 * M
      rb1 = jnp.zeros((L,), jnp.int32) + r1 * M
      for v in range(NVEC):
        v00 = plsc.load_gather(img, [rb0 + c0[v]])
        v01 = plsc.load_gather(img, [rb0 + c1[v]])
        v10 = plsc.load_gather(img, [rb1 + c0[v]])
        v11 = plsc.load_gather(img, [rb1 + c1[v]])
        top = v00 + fx * (v01 - v00)
        bot = v10 + fx * (v11 - v10)
        ob[pl.ds(i * N + v * L, L)] = top + fy * (bot - top)

  def pair_body(p, carry):
    for k, img, ob, isem, osem in ((0, img0, ob0, is0, os0),
                                   (1, img1, ob1, is1, os1)):
      g = 2 * p + k
      b = base + g
      pltpu.make_async_copy(obj_hbm.at[b], img, isem).wait()

      @pl.when(p > 0)
      def _():
        pltpu.make_async_copy(ob, out_hbm.at[b - 2], osem).wait()

      compute_batch(g, img, ob)
      pltpu.async_copy(ob, out_hbm.at[b], osem)

      @pl.when(p < BPW // 2 - 1)
      def _():
        pltpu.async_copy(obj_hbm.at[b + 2], img, isem)
    return carry

  lax.fori_loop(0, BPW // 2, pair_body, 0)
  pltpu.make_async_copy(ob0, out_hbm.at[base + BPW - 2], os0).wait()
  pltpu.make_async_copy(ob1, out_hbm.at[base + BPW - 1], os1).wait()


@jax.jit
def _run(obj, px, py):
  kern = pl.kernel(
      _patch_body,
      out_type=jax.ShapeDtypeStruct((B, N * N), jnp.float32),
      mesh=plsc.VectorSubcoreMesh(core_axis_name="c", subcore_axis_name="s"),
      compiler_params=pltpu.CompilerParams(needs_layout_passes=False),
      scratch_types=[
          pltpu.VMEM((M * M,), jnp.float32),
          pltpu.VMEM((M * M,), jnp.float32),
          pltpu.VMEM((N * N,), jnp.float32),
          pltpu.VMEM((N * N,), jnp.float32),
          pltpu.VMEM((BPW,), jnp.float32),
          pltpu.VMEM((BPW,), jnp.float32),
          pltpu.SemaphoreType.DMA,
          pltpu.SemaphoreType.DMA,
          pltpu.SemaphoreType.DMA,
          pltpu.SemaphoreType.DMA,
      ],
  )
  return kern(obj, px, py)


def kernel(padded_obj, positions):
  obj = padded_obj.reshape(B, M * M)
  px = positions[:, 0]
  py = positions[:, 1]
  out = _run(obj, px, py)
  return out.reshape(B, N, N, 1)


# speedup vs baseline: 6.0185x; 1.0364x over previous
import jax
import jax.numpy as jnp
from jax import lax
from jax.experimental import pallas as pl
from jax.experimental.pallas import tpu as pltpu
from jax.experimental.pallas import tpu_sc as plsc

B = 4096
M = 74
N = 64
HALF = 5.0

NC = 2
NS = 16
NW = NC * NS
BPW = B // NW
L = 16
NVEC = N // L


def _floor_f32(x):
  t = x.astype(jnp.int32)
  return t - (t.astype(jnp.float32) > x).astype(jnp.int32)


def _patch_body(obj_hbm, px_hbm, py_hbm, out_hbm,
                img0, img1, ob0, ob1, pxl, pyl,
                is0, is1, os0, os1):
  wid = lax.axis_index("s") * NC + lax.axis_index("c")
  base = wid * BPW
  pltpu.sync_copy(px_hbm.at[pl.ds(base, BPW)], pxl)
  pltpu.sync_copy(py_hbm.at[pl.ds(base, BPW)], pyl)
  iot = lax.iota(jnp.int32, 16)

  pltpu.async_copy(obj_hbm.at[base], img0, is0)
  pltpu.async_copy(obj_hbm.at[base + 1], img1, is1)

  def compute_batch(g, img, ob):
    grp = (g // L) * L
    lane = g % L
    oh = iot == lane
    pxv = pxl[pl.ds(grp, L)] + HALF
    pyv = pyl[pl.ds(grp, L)] + HALF
    oxv = _floor_f32(pxv)
    oyv = _floor_f32(pyv)
    fxv = pxv - oxv.astype(jnp.float32)
    fyv = pyv - oyv.astype(jnp.float32)
    ox = jnp.sum(jnp.where(oh, oxv, 0))
    oy = jnp.sum(jnp.where(oh, oyv, 0))
    fx = jnp.sum(jnp.where(oh, fxv, 0.0))
    fy = jnp.sum(jnp.where(oh, fyv, 0.0))

    c0 = [jnp.clip(iot + (v * L) + ox, 0, M - 1) for v in range(NVEC)]
    c1 = [jnp.minimum(c + 1, M - 1) for c in c0]

    def fast_path():
      rb0 = jnp.zeros((L,), jnp.int32) + jnp.minimum(oy, M - 1) * M
      tops = []
      for v in range(NVEC):
        v00 = plsc.load_gather(img, [rb0 + c0[v]])
        v01 = plsc.load_gather(img, [rb0 + c1[v]])
        tops.append(v00 + fx * (v01 - v00))

      @plsc.parallel_loop(0, N, 1, unroll=4, carry=tuple(tops))
      def row_body(i, tops_c):
        r1 = jnp.minimum(jnp.minimum(i + oy, M - 1) + 1, M - 1)
        rb1 = jnp.zeros((L,), jnp.int32) + r1 * M
        bots = []
        for v in range(NVEC):
          v10 = plsc.load_gather(img, [rb1 + c0[v]])
          v11 = plsc.load_gather(img, [rb1 + c1[v]])
          bot = v10 + fx * (v11 - v10)
          ob[pl.ds(i * N + v * L, L)] = tops_c[v] + fy * (bot - tops_c[v])
          bots.append(bot)
        return tuple(bots)

    def slow_path():
      @plsc.parallel_loop(0, N, 1, unroll=4)
      def row_body(i):
        r0 = jnp.clip(i + oy, 0, M - 1)
        r1 = jnp.minimum(r0 + 1, M - 1)
        rb0 = jnp.zeros((L,), jnp.int32) + r0 * M
        rb1 = jnp.zeros((L,), jnp.int32) + r1 * M
        for v in range(NVEC):
          v00 = plsc.load_gather(img, [rb0 + c0[v]])
          v01 = plsc.load_gather(img, [rb0 + c1[v]])
          v10 = plsc.load_gather(img, [rb1 + c0[v]])
          v11 = plsc.load_gather(img, [rb1 + c1[v]])
          top = v00 + fx * (v01 - v00)
          bot = v10 + fx * (v11 - v10)
          ob[pl.ds(i * N + v * L, L)] = top + fy * (bot - top)

    lax.cond(oy >= 0, fast_path, slow_path)

  def pair_body(p, carry):
    for k, img, ob, isem, osem in ((0, img0, ob0, is0, os0),
                                   (1, img1, ob1, is1, os1)):
      g = 2 * p + k
      b = base + g
      pltpu.make_async_copy(obj_hbm.at[b], img, isem).wait()

      @pl.when(p > 0)
      def _():
        pltpu.make_async_copy(ob, out_hbm.at[b - 2], osem).wait()

      compute_batch(g, img, ob)
      pltpu.async_copy(ob, out_hbm.at[b], osem)

      @pl.when(p < BPW // 2 - 1)
      def _():
        pltpu.async_copy(obj_hbm.at[b + 2], img, isem)
    return carry

  lax.fori_loop(0, BPW // 2, pair_body, 0)
  pltpu.make_async_copy(ob0, out_hbm.at[base + BPW - 2], os0).wait()
  pltpu.make_async_copy(ob1, out_hbm.at[base + BPW - 1], os1).wait()


@jax.jit
def _run(obj, px, py):
  kern = pl.kernel(
      _patch_body,
      out_type=jax.ShapeDtypeStruct((B, N * N), jnp.float32),
      mesh=plsc.VectorSubcoreMesh(core_axis_name="c", subcore_axis_name="s"),
      compiler_params=pltpu.CompilerParams(needs_layout_passes=False),
      scratch_types=[
          pltpu.VMEM((M * M,), jnp.float32),
          pltpu.VMEM((M * M,), jnp.float32),
          pltpu.VMEM((N * N,), jnp.float32),
          pltpu.VMEM((N * N,), jnp.float32),
          pltpu.VMEM((BPW,), jnp.float32),
          pltpu.VMEM((BPW,), jnp.float32),
          pltpu.SemaphoreType.DMA,
          pltpu.SemaphoreType.DMA,
          pltpu.SemaphoreType.DMA,
          pltpu.SemaphoreType.DMA,
      ],
  )
  return kern(obj, px, py)


def kernel(padded_obj, positions):
  obj = padded_obj.reshape(B, M * M)
  px = positions[:, 0]
  py = positions[:, 1]
  out = _run(obj, px, py)
  return out.reshape(B, N, N, 1)
